# baseline (device time: 56429 ns/iter reference)
import jax
import jax.numpy as jnp
from jax import lax
from jax.experimental import pallas as pl
from jax.experimental.pallas import tpu as pltpu

N_DEV = 4
N_LAYERS = 3


def kernel(x, Win0, Wout0, Win1, Wout1, Win2, Wout2):
    b, _ = x.shape
    h_dim = Win0.shape[1]
    out_cols = Wout0.shape[1]

    def body(x_ref, win0, wout0, win1, wout1, win2, wout2,
             out_ref, comm_ref, send_sems, recv_sems):
        my = lax.axis_index("i")
        left = (my + N_DEV - 1) % N_DEV
        right = (my + 1) % N_DEV

        barrier_sem = pltpu.get_barrier_semaphore()
        for nbr in (left, right):
            pl.semaphore_signal(
                barrier_sem, inc=1,
                device_id=(nbr,), device_id_type=pl.DeviceIdType.MESH,
            )
        pl.semaphore_wait(barrier_sem, 2)

        wins = (win0, win1, win2)
        wouts = (wout0, wout1, wout2)

        x_cur = x_ref[:, :]
        for l in range(N_LAYERS):
            partial = jnp.dot(
                x_cur, wins[l][:, :], preferred_element_type=jnp.float32
            )
            comm_ref[l, 0, :, :] = partial
            acc = partial
            for h in range(N_DEV - 1):
                rdma = pltpu.make_async_remote_copy(
                    src_ref=comm_ref.at[l, h],
                    dst_ref=comm_ref.at[l, h + 1],
                    send_sem=send_sems.at[l, h],
                    recv_sem=recv_sems.at[l, h],
                    device_id=(right,),
                    device_id_type=pl.DeviceIdType.MESH,
                )
                rdma.start()
                rdma.wait()
                acc = acc + comm_ref[l, h + 1, :, :]
            hidden = jnp.maximum(acc, 0.0)
            x_cur = jnp.dot(
                hidden, wouts[l][:, :], preferred_element_type=jnp.float32
            )
        out_ref[:, :] = x_cur

    return pl.pallas_call(
        body,
        out_shape=jax.ShapeDtypeStruct((b, out_cols), jnp.float32),
        in_specs=[pl.BlockSpec(memory_space=pltpu.VMEM)] * 7,
        out_specs=pl.BlockSpec(memory_space=pltpu.VMEM),
        scratch_shapes=[
            pltpu.VMEM((N_LAYERS, N_DEV, b, h_dim), jnp.float32),
            pltpu.SemaphoreType.DMA((N_LAYERS, N_DEV - 1)),
            pltpu.SemaphoreType.DMA((N_LAYERS, N_DEV - 1)),
        ],
        compiler_params=pltpu.CompilerParams(collective_id=0),
    )(x, Win0, Wout0, Win1, Wout1, Win2, Wout2)


# device time: 39120 ns/iter; 1.4425x vs baseline; 1.4425x over previous
import jax
import jax.numpy as jnp
from jax import lax
from jax.experimental import pallas as pl
from jax.experimental.pallas import tpu as pltpu

N_DEV = 4
N_LAYERS = 3


def kernel(x, Win0, Wout0, Win1, Wout1, Win2, Wout2):
    b, _ = x.shape
    h_dim = Win0.shape[1]
    out_cols = Wout0.shape[1]

    def body(x_ref, win0, wout0, win1, wout1, win2, wout2,
             out_ref, comm_ref, send_sems, recv_sems):
        my = lax.axis_index("i")

        barrier_sem = pltpu.get_barrier_semaphore()
        for d in range(1, N_DEV):
            pl.semaphore_signal(
                barrier_sem, inc=1,
                device_id=((my + d) % N_DEV,),
                device_id_type=pl.DeviceIdType.MESH,
            )
        pl.semaphore_wait(barrier_sem, N_DEV - 1)

        wins = (win0, win1, win2)
        wouts = (wout0, wout1, wout2)

        x_cur = x_ref[:, :]
        for l in range(N_LAYERS):
            partial = jnp.dot(
                x_cur, wins[l][:, :], preferred_element_type=jnp.float32
            )
            comm_ref[l, 0, :, :] = partial
            rdmas = []
            for d in range(1, N_DEV):
                rdma = pltpu.make_async_remote_copy(
                    src_ref=comm_ref.at[l, 0],
                    dst_ref=comm_ref.at[l, d],
                    send_sem=send_sems.at[l, d - 1],
                    recv_sem=recv_sems.at[l, d - 1],
                    device_id=((my + d) % N_DEV,),
                    device_id_type=pl.DeviceIdType.MESH,
                )
                rdma.start()
                rdmas.append(rdma)
            acc = partial
            for d in range(1, N_DEV):
                rdmas[d - 1].wait_recv()
                acc = acc + comm_ref[l, d, :, :]
            hidden = jnp.maximum(acc, 0.0)
            x_cur = jnp.dot(
                hidden, wouts[l][:, :], preferred_element_type=jnp.float32
            )
            for d in range(1, N_DEV):
                rdmas[d - 1].wait_send()
        out_ref[:, :] = x_cur

    return pl.pallas_call(
        body,
        out_shape=jax.ShapeDtypeStruct((b, out_cols), jnp.float32),
        in_specs=[pl.BlockSpec(memory_space=pltpu.VMEM)] * 7,
        out_specs=pl.BlockSpec(memory_space=pltpu.VMEM),
        scratch_shapes=[
            pltpu.VMEM((N_LAYERS, N_DEV, b, h_dim), jnp.float32),
            pltpu.SemaphoreType.DMA((N_LAYERS, N_DEV - 1)),
            pltpu.SemaphoreType.DMA((N_LAYERS, N_DEV - 1)),
        ],
        compiler_params=pltpu.CompilerParams(collective_id=0),
    )(x, Win0, Wout0, Win1, Wout1, Win2, Wout2)


# device time: 38763 ns/iter; 1.4557x vs baseline; 1.0092x over previous
import jax
import jax.numpy as jnp
from jax import lax
from jax.experimental import pallas as pl
from jax.experimental.pallas import tpu as pltpu

N_DEV = 4
N_LAYERS = 3
CHUNKS = 4


def kernel(x, Win0, Wout0, Win1, Wout1, Win2, Wout2):
    b, _ = x.shape
    h_dim = Win0.shape[1]
    out_cols = Wout0.shape[1]
    cw = h_dim // CHUNKS

    def body(x_ref, win0, wout0, win1, wout1, win2, wout2,
             out_ref, comm_ref, send_sems, recv_sems):
        my = lax.axis_index("i")

        barrier_sem = pltpu.get_barrier_semaphore()
        for d in range(1, N_DEV):
            pl.semaphore_signal(
                barrier_sem, inc=1,
                device_id=((my + d) % N_DEV,),
                device_id_type=pl.DeviceIdType.MESH,
            )
        pl.semaphore_wait(barrier_sem, N_DEV - 1)

        wins = (win0, win1, win2)
        wouts = (wout0, wout1, wout2)

        def make_rdma(l, d, c):
            return pltpu.make_async_remote_copy(
                src_ref=comm_ref.at[l, 0, c],
                dst_ref=comm_ref.at[l, d, c],
                send_sem=send_sems.at[l, d - 1, c],
                recv_sem=recv_sems.at[l, d - 1, c],
                device_id=((my + d) % N_DEV,),
                device_id_type=pl.DeviceIdType.MESH,
            )

        x_cur = x_ref[:, :]
        for l in range(N_LAYERS):
            rdmas = []
            for c in range(CHUNKS):
                partial_c = jnp.dot(
                    x_cur, wins[l][:, c * cw:(c + 1) * cw],
                    preferred_element_type=jnp.float32,
                )
                comm_ref[l, 0, c, :, :] = partial_c
                for d in range(1, N_DEV):
                    rdma = make_rdma(l, d, c)
                    rdma.start()
                    rdmas.append(rdma)
            x_next = None
            for c in range(CHUNKS):
                acc = comm_ref[l, 0, c, :, :]
                for d in range(1, N_DEV):
                    rdmas[c * (N_DEV - 1) + (d - 1)].wait_recv()
                    acc = acc + comm_ref[l, d, c, :, :]
                h_c = jnp.maximum(acc, 0.0)
                part = jnp.dot(
                    h_c, wouts[l][c * cw:(c + 1) * cw, :],
                    preferred_element_type=jnp.float32,
                )
                x_next = part if x_next is None else x_next + part
            x_cur = x_next
            for r in rdmas:
                r.wait_send()
        out_ref[:, :] = x_cur

    return pl.pallas_call(
        body,
        out_shape=jax.ShapeDtypeStruct((b, out_cols), jnp.float32),
        in_specs=[pl.BlockSpec(memory_space=pltpu.VMEM)] * 7,
        out_specs=pl.BlockSpec(memory_space=pltpu.VMEM),
        scratch_shapes=[
            pltpu.VMEM((N_LAYERS, N_DEV, CHUNKS, b, cw), jnp.float32),
            pltpu.SemaphoreType.DMA((N_LAYERS, N_DEV - 1, CHUNKS)),
            pltpu.SemaphoreType.DMA((N_LAYERS, N_DEV - 1, CHUNKS)),
        ],
        compiler_params=pltpu.CompilerParams(collective_id=0),
    )(x, Win0, Wout0, Win1, Wout1, Win2, Wout2)


# device time: 30349 ns/iter; 1.8593x vs baseline; 1.2772x over previous
import jax
import jax.numpy as jnp
from jax import lax
from jax.experimental import pallas as pl
from jax.experimental.pallas import tpu as pltpu

N_DEV = 4
N_LAYERS = 3
CHUNKS = 4


def kernel(x, Win0, Wout0, Win1, Wout1, Win2, Wout2):
    b, _ = x.shape
    h_dim = Win0.shape[1]
    out_cols = Wout0.shape[1]
    cw = h_dim // CHUNKS

    def body(x_ref, win0, wout0, win1, wout1, win2, wout2,
             out_ref, comm_ref, send_sems, recv_sems):
        my = lax.axis_index("i")

        barrier_sem = pltpu.get_barrier_semaphore()
        for d in range(1, N_DEV):
            pl.semaphore_signal(
                barrier_sem, inc=1,
                device_id=((my + d) % N_DEV,),
                device_id_type=pl.DeviceIdType.MESH,
            )
        pl.semaphore_wait(barrier_sem, N_DEV - 1)

        wins = (win0, win1, win2)
        wouts = (wout0, wout1, wout2)

        def make_rdma(l, d, c):
            return pltpu.make_async_remote_copy(
                src_ref=comm_ref.at[l, 0, c],
                dst_ref=comm_ref.at[l, d, c],
                send_sem=send_sems.at[l, d - 1, c],
                recv_sem=recv_sems.at[l, d - 1, c],
                device_id=((my + d) % N_DEV,),
                device_id_type=pl.DeviceIdType.MESH,
            )

        x_cur = x_ref[:, :]
        for l in range(N_LAYERS):
            rdmas = []
            for c in range(CHUNKS):
                partial_c = jnp.dot(
                    x_cur, wins[l][:, c * cw:(c + 1) * cw],
                    preferred_element_type=jnp.float32,
                )
                comm_ref[l, 0, c, :, :] = partial_c.astype(jnp.bfloat16)
                for d in range(1, N_DEV):
                    rdma = make_rdma(l, d, c)
                    rdma.start()
                    rdmas.append(rdma)
            x_next = None
            for c in range(CHUNKS):
                acc = comm_ref[l, 0, c, :, :].astype(jnp.float32)
                for d in range(1, N_DEV):
                    rdmas[c * (N_DEV - 1) + (d - 1)].wait_recv()
                    acc = acc + comm_ref[l, d, c, :, :].astype(jnp.float32)
                h_c = jnp.maximum(acc, 0.0)
                part = jnp.dot(
                    h_c, wouts[l][c * cw:(c + 1) * cw, :],
                    preferred_element_type=jnp.float32,
                )
                x_next = part if x_next is None else x_next + part
            x_cur = x_next
            for r in rdmas:
                r.wait_send()
        out_ref[:, :] = x_cur

    return pl.pallas_call(
        body,
        out_shape=jax.ShapeDtypeStruct((b, out_cols), jnp.float32),
        in_specs=[pl.BlockSpec(memory_space=pltpu.VMEM)] * 7,
        out_specs=pl.BlockSpec(memory_space=pltpu.VMEM),
        scratch_shapes=[
            pltpu.VMEM((N_LAYERS, N_DEV, CHUNKS, b, cw), jnp.bfloat16),
            pltpu.SemaphoreType.DMA((N_LAYERS, N_DEV - 1, CHUNKS)),
            pltpu.SemaphoreType.DMA((N_LAYERS, N_DEV - 1, CHUNKS)),
        ],
        compiler_params=pltpu.CompilerParams(collective_id=0),
    )(x, Win0, Wout0, Win1, Wout1, Win2, Wout2)
